# baseline (speedup 1.0000x reference)
import jax
import jax.numpy as jnp
from jax import lax
from jax.experimental import pallas as pl
from jax.experimental.pallas import tpu as pltpu
from jax.experimental.pallas import tpu_sc as plsc

NE = 100000
NR = 1000
D = 128
B = 16384

NC = 2
NS = 16
L = 16
NW = NC * NS
PAIR = 64
NGRP = PAIR // L
RB = 32
POSM = 0xFFFFF


def _rsqrt(p):
    i = plsc.bitcast(p, jnp.int32)
    i = jnp.int32(0x5F3759DF) - lax.shift_right_logical(i, 1)
    y = plsc.bitcast(i, jnp.float32)
    for _ in range(4):
        y = y * (1.5 - 0.5 * p * y * y)
    return y


def _sc_body(u_idx, v_idx, r_idx, E, Wu, rv, bs, bo, E1, rv1, out,
             r_copy, posp, posg, u_r, v_r, u1_r, v1_r, wu_l, rv_l, rv1_l,
             iu, iv, bs_b, bo_b, pos2d, val, semI, semR, semO):
    wid = lax.axis_index("s") * NC + lax.axis_index("c")
    lo = lax.div(wid * NR, NW)
    hi = lax.div((wid + 1) * NR, NW)
    lane = lax.iota(jnp.int32, 16)
    zero = jnp.zeros((16,), jnp.float32)

    crel = [pltpu.async_copy(Wu.at[pl.ds(lo, RB)], wu_l, semR.at[0]),
            pltpu.async_copy(rv.at[pl.ds(lo, RB)], rv_l, semR.at[0]),
            pltpu.async_copy(rv1.at[pl.ds(lo, RB)], rv1_l, semR.at[0])]

    pltpu.sync_copy(r_idx, r_copy)

    def scan_body(i, cnt):
        rvec = r_copy[pl.ds(i * 16, 16)]
        m = (rvec >= lo) & (rvec < hi)
        packed = (lane + i * 16) | lax.shift_left(rvec - lo, 20)
        plsc.store_compressed(posp.at[pl.ds(cnt, 16)], packed, mask=m)
        return cnt + jnp.max(plsc.all_reduce_population_count(m))

    n = lax.fori_loop(0, B // 16, scan_body, jnp.int32(0))
    npairs = lax.div(n + PAIR - 1, PAIR)

    @pl.when(n > 0)
    def _():
        last = jnp.full((16,), posp[pl.ds(n - 1, 16)][0], jnp.int32)
        for k in range(NGRP):
            posp[pl.ds(n + k * 16, 16)] = last

    def clean_body(i, _):
        posg[pl.ds(i * 16, 16)] = posp[pl.ds(i * 16, 16)] & POSM
        return 0

    lax.fori_loop(0, npairs * NGRP, clean_body, 0)

    for cp in crel:
        cp.wait()

    def idxg_refs(t, p):
        sl = pl.ds(t * PAIR, PAIR)
        return ((u_idx.at[posg.at[sl]], iu.at[p]),
                (v_idx.at[posg.at[sl]], iv.at[p]))

    def issue_idxg(t, p):
        for s, d in idxg_refs(t, p):
            pltpu.async_copy(s, d, semI.at[p])

    def drain_idxg(t, p):
        for s, d in idxg_refs(t, p):
            pltpu.make_async_copy(s, d, semI.at[p]).wait()

    def rowg_refs(p):
        return ((E.at[iu.at[p]], u_r.at[p]),
                (E.at[iv.at[p]], v_r.at[p]),
                (E1.at[iu.at[p]], u1_r.at[p]),
                (E1.at[iv.at[p]], v1_r.at[p]),
                (bs.at[iu.at[p]], bs_b.at[p]),
                (bo.at[iv.at[p]], bo_b.at[p]))

    def issue_rowg(p):
        for s, d in rowg_refs(p):
            pltpu.async_copy(s, d, semR.at[p])

    def drain_rowg(p):
        for s, d in rowg_refs(p):
            pltpu.make_async_copy(s, d, semR.at[p]).wait()

    def drain_scat(p):
        pltpu.make_async_copy(val.at[p], out.at[pos2d.at[p]],
                              semO.at[p]).wait()

    def compute(t, p):
        @pl.when(t >= 2)
        def _():
            drain_scat(p)

        for g in range(NGRP):
            posv = posp[pl.ds(t * PAIR + g * L, L)]
            rl = lax.shift_right_logical(posv, 20)
            rows = lane + g * L

            def dbody(d, carry):
                su, sa, dot, e = carry
                col = jnp.bitwise_and(d + lane, D - 1)
                u = plsc.load_gather(u_r.at[p], [rows, col])
                ru = plsc.load_gather(wu_l, [rl, col])
                uw = u * ru
                v = plsc.load_gather(v_r.at[p], [rows, col])
                rvv = plsc.load_gather(rv_l, [rl, col])
                a = v + rvv
                su = su + uw * uw
                sa = sa + a * a
                dot = dot + uw * a
                u1 = plsc.load_gather(u1_r.at[p], [rows, col])
                v1 = plsc.load_gather(v1_r.at[p], [rows, col])
                rv1v = plsc.load_gather(rv1_l, [rl, col])
                d1 = u1 * ru - v1 - rv1v
                e = e + d1 * d1
                return (su, sa, dot, e)

            su, sa, dot, e = lax.fori_loop(
                0, D, dbody, (zero, zero, zero, zero))
            q = (su + 1.0) * (sa + 1.0)
            sq = q * _rsqrt(q)
            res = (2.0 + 2.0 * dot - 2.0 * sq - e
                   + bs_b[p, pl.ds(g * L, L)] + bo_b[p, pl.ds(g * L, L)])
            pos2d[p, pl.ds(g * L, L)] = posv & POSM
            val[p, pl.ds(g * L, L)] = res

        pltpu.async_copy(val.at[p], out.at[pos2d.at[p]], semO.at[p])

    def step(t, p):
        q = 1 - p

        @pl.when(t + 1 < npairs)
        def _():
            drain_idxg(t + 1, q)
            issue_rowg(q)

        drain_rowg(p)

        @pl.when(t + 2 < npairs)
        def _():
            issue_idxg(t + 2, p)

        compute(t, p)

    @pl.when(npairs > 0)
    def _():
        issue_idxg(0, 0)
        drain_idxg(0, 0)
        issue_rowg(0)

        @pl.when(npairs > 1)
        def _():
            issue_idxg(1, 1)

    def pp_body(u, _):
        t0 = 2 * u
        step(t0, 0)

        @pl.when(t0 + 1 < npairs)
        def _():
            step(t0 + 1, 1)

        return 0

    lax.fori_loop(0, lax.div(npairs + 1, 2), pp_body, 0)

    @pl.when(npairs >= 1)
    def _():
        drain_scat(0)

    @pl.when(npairs >= 2)
    def _():
        drain_scat(1)


@jax.jit
def _mu_rel_sc(u_idx, r_idx, v_idx, E, Wu, rv, bs, bo, E1, rv1):
    mesh = plsc.VectorSubcoreMesh(core_axis_name="c", subcore_axis_name="s")
    kern = pl.kernel(
        _sc_body,
        out_type=jax.ShapeDtypeStruct((B,), jnp.float32),
        mesh=mesh,
        scratch_types=[
            pltpu.VMEM((B,), jnp.int32),
            pltpu.VMEM((B + PAIR,), jnp.int32),
            pltpu.VMEM((B + PAIR,), jnp.int32),
            pltpu.VMEM((2, PAIR, D), jnp.float32),
            pltpu.VMEM((2, PAIR, D), jnp.float32),
            pltpu.VMEM((2, PAIR, D), jnp.float32),
            pltpu.VMEM((2, PAIR, D), jnp.float32),
            pltpu.VMEM((RB, D), jnp.float32),
            pltpu.VMEM((RB, D), jnp.float32),
            pltpu.VMEM((RB, D), jnp.float32),
            pltpu.VMEM((2, PAIR), jnp.int32),
            pltpu.VMEM((2, PAIR), jnp.int32),
            pltpu.VMEM((2, PAIR), jnp.float32),
            pltpu.VMEM((2, PAIR), jnp.float32),
            pltpu.VMEM((2, PAIR), jnp.int32),
            pltpu.VMEM((2, PAIR), jnp.float32),
            pltpu.SemaphoreType.DMA((2,)),
            pltpu.SemaphoreType.DMA((2,)),
            pltpu.SemaphoreType.DMA((2,)),
        ],
        compiler_params=pltpu.CompilerParams(
            use_tc_tiling_on_sc=False, needs_layout_passes=False,
            skip_device_barrier=True, disable_bounds_checks=True),
    )
    return kern(u_idx, v_idx, r_idx, E, Wu, rv, bs, bo, E1, rv1)


def kernel(u_idx, r_idx, v_idx, E, Wu, rv, bs, bo, E1, Wu1, rv1):
    del Wu1
    return _mu_rel_sc(u_idx, r_idx, v_idx, E, Wu, rv, bs, bo, E1, rv1)


# speedup vs baseline: 4.7430x; 4.7430x over previous
import functools

import jax
import jax.numpy as jnp
from jax import lax
from jax.experimental import pallas as pl
from jax.experimental.pallas import tpu as pltpu
from jax.experimental.pallas import tpu_sc as plsc

NE = 100000
NR = 1000
D = 128
B = 16384

NC = 2
NS = 16
L = 16
NW = NC * NS
BPW = B // NW
CH = 64
NCHUNK = BPW // CH
NGRP = CH // L


def _rsqrt(p):
    i = plsc.bitcast(p, jnp.int32)
    i = jnp.int32(0x5F3759DF) - lax.shift_right_logical(i, 1)
    y = plsc.bitcast(i, jnp.float32)
    for _ in range(4):
        y = y * (1.5 - 0.5 * p * y * y)
    return y


def _sc_body(u_idx, v_idx, r_idx, E, Wu, rv, bs, bo, E1, rv1, out,
             idx_u, idx_v, idx_r, bs_b, bo_b,
             u_r, v_r, u1_r, v1_r, ru_r, rvv_r, rv1_r, out_v, sem):
    sid = lax.axis_index("s")
    wid = sid * NC + lax.axis_index("c")
    base = wid * BPW
    lane = lax.iota(jnp.int32, 16)
    zero = jnp.zeros((16,), jnp.float32)

    pltpu.sync_copy(u_idx.at[pl.ds(base, BPW)], idx_u)
    pltpu.sync_copy(v_idx.at[pl.ds(base, BPW)], idx_v)
    pltpu.sync_copy(r_idx.at[pl.ds(base, BPW)], idx_r)


    def bufs(p):
        return (u_r.at[p], v_r.at[p], u1_r.at[p], v1_r.at[p],
                ru_r.at[p], rvv_r.at[p], rv1_r.at[p],
                bs_b.at[p], bo_b.at[p])

    def srcs(c):
        iu = idx_u.at[pl.ds(c * CH, CH)]
        iv = idx_v.at[pl.ds(c * CH, CH)]
        ir = idx_r.at[pl.ds(c * CH, CH)]
        return (E.at[iu], E.at[iv], E1.at[iu], E1.at[iv],
                Wu.at[ir], rv.at[ir], rv1.at[ir],
                bs.at[iu], bo.at[iv])

    def issue(c, p):
        for s, b in zip(srcs(c), bufs(p)):
            pltpu.async_copy(s, b, sem.at[p])

    def drain(c, p):
        for s, b in zip(srcs(c), bufs(p)):
            pltpu.make_async_copy(s, b, sem.at[p]).wait()

    def compute(c, p):
        u_b, v_b, u1_b, v1_b, ru_b, rvv_b, rv1_b, bs_bp, bo_bp = bufs(p)

        def gbody(g, _):
            rows = lane + g * L

            def dbody(d, carry):
                su, sa, dot, e = carry
                col = jnp.bitwise_and(d + lane, D - 1)
                u = plsc.load_gather(u_b, [rows, col])
                ru = plsc.load_gather(ru_b, [rows, col])
                uw = u * ru
                v = plsc.load_gather(v_b, [rows, col])
                rvv = plsc.load_gather(rvv_b, [rows, col])
                a = v + rvv
                su = su + uw * uw
                sa = sa + a * a
                dot = dot + uw * a
                u1 = plsc.load_gather(u1_b, [rows, col])
                v1 = plsc.load_gather(v1_b, [rows, col])
                rv1v = plsc.load_gather(rv1_b, [rows, col])
                d1 = u1 * ru - v1 - rv1v
                e = e + d1 * d1
                return (su, sa, dot, e)

            su, sa, dot, e = lax.fori_loop(
                0, D, dbody, (zero, zero, zero, zero))
            q = (su + 1.0) * (sa + 1.0)
            sq = q * _rsqrt(q)
            res = (2.0 + 2.0 * dot - 2.0 * sq - e
                   + bs_bp[pl.ds(g * L, L)] + bo_bp[pl.ds(g * L, L)])
            out_v[pl.ds(c * CH + g * L, L)] = res
            return 0

        lax.fori_loop(0, NGRP, gbody, 0)

    issue(0, 0)

    def pair_body(t, _):
        c0 = 2 * t
        c1 = c0 + 1
        issue(c1, 1)
        drain(c0, 0)
        compute(c0, 0)

        @pl.when(t < NCHUNK // 2 - 1)
        def _():
            issue(c0 + 2, 0)

        drain(c1, 1)
        compute(c1, 1)
        return 0

    lax.fori_loop(0, NCHUNK // 2, pair_body, 0)

    pltpu.sync_copy(out_v, out.at[pl.ds(base, BPW)])


@jax.jit
def _mu_rel_sc(u_idx, r_idx, v_idx, E, Wu, rv, bs, bo, E1, rv1):
    mesh = plsc.VectorSubcoreMesh(core_axis_name="c", subcore_axis_name="s")
    kern = pl.kernel(
        _sc_body,
        out_type=jax.ShapeDtypeStruct((B,), jnp.float32),
        mesh=mesh,
        scratch_types=[
            pltpu.VMEM((BPW,), jnp.int32),
            pltpu.VMEM((BPW,), jnp.int32),
            pltpu.VMEM((BPW,), jnp.int32),
            pltpu.VMEM((2, CH), jnp.float32),
            pltpu.VMEM((2, CH), jnp.float32),
            pltpu.VMEM((2, CH, D), jnp.float32),
            pltpu.VMEM((2, CH, D), jnp.float32),
            pltpu.VMEM((2, CH, D), jnp.float32),
            pltpu.VMEM((2, CH, D), jnp.float32),
            pltpu.VMEM((2, CH, D), jnp.float32),
            pltpu.VMEM((2, CH, D), jnp.float32),
            pltpu.VMEM((2, CH, D), jnp.float32),
            pltpu.VMEM((BPW,), jnp.float32),
            pltpu.SemaphoreType.DMA((2,)),
        ],
        compiler_params=pltpu.CompilerParams(
            use_tc_tiling_on_sc=False, needs_layout_passes=False,
            skip_device_barrier=True, disable_bounds_checks=True),
    )
    return kern(u_idx, v_idx, r_idx, E, Wu, rv, bs, bo, E1, rv1)


def kernel(u_idx, r_idx, v_idx, E, Wu, rv, bs, bo, E1, Wu1, rv1):
    del Wu1
    return _mu_rel_sc(u_idx, r_idx, v_idx, E, Wu, rv, bs, bo, E1, rv1)
